# baseline (speedup 1.0000x reference)
import functools

import jax
import jax.numpy as jnp
from jax import lax
from jax.experimental import pallas as pl
from jax.experimental.pallas import tpu as pltpu
from jax.experimental.pallas import tpu_sc as plsc

_D = 32
_NC, _NS = 2, 16
_NW = _NC * _NS


@functools.partial(jax.jit, static_argnums=(2, 3))
def _sc_gather_t(idx2d, table, hist, nb):
    cb = nb // _NW
    mesh = plsc.VectorSubcoreMesh(core_axis_name="c", subcore_axis_name="s")

    @functools.partial(
        pl.kernel,
        out_type=jax.ShapeDtypeStruct((hist, _D, nb), jnp.float32),
        mesh=mesh,
        compiler_params=pltpu.CompilerParams(use_tc_tiling_on_sc=False,
                                             needs_layout_passes=False),
        scratch_types=[
            pltpu.VMEM((2, cb), jnp.int32),
            pltpu.VMEM((2, cb, _D), jnp.float32),
            pltpu.VMEM((2, _D, cb), jnp.float32),
            pltpu.SemaphoreType.DMA,
            pltpu.SemaphoreType.DMA,
            pltpu.SemaphoreType.DMA,
            pltpu.SemaphoreType.DMA,
            pltpu.SemaphoreType.DMA,
            pltpu.SemaphoreType.DMA,
        ],
    )
    def k(idx_hbm, table_hbm, out_hbm, idx_v, rows_v, trows_v,
          i0, i1, g0, g1, s0, s1):
        wid = lax.axis_index("s") * _NC + lax.axis_index("c")
        b0 = pl.multiple_of(wid * cb, cb)
        isems = (i0, i1)
        gsems = (g0, g1)
        ssems = (s0, s1)
        lanes = lax.iota(jnp.int32, 16)
        dcols = [jnp.full((16,), d, jnp.int32) for d in range(_D)]

        def idx_load(h, b):
            pltpu.async_copy(idx_hbm.at[h, pl.ds(b0, cb)], idx_v.at[b], isems[b])

        def idx_wait(h, b):
            pltpu.make_async_copy(idx_hbm.at[h, pl.ds(b0, cb)], idx_v.at[b],
                                  isems[b]).wait()

        def gather_start(b):
            pltpu.async_copy(table_hbm.at[idx_v.at[b]], rows_v.at[b], gsems[b])

        def gather_wait(b):
            pltpu.make_async_copy(table_hbm.at[idx_v.at[b]], rows_v.at[b],
                                  gsems[b]).wait()

        def store_start(h, b):
            pltpu.async_copy(trows_v.at[b], out_hbm.at[h, :, pl.ds(b0, cb)],
                             ssems[b])

        def store_wait(h, b):
            pltpu.make_async_copy(trows_v.at[b], out_hbm.at[h, :, pl.ds(b0, cb)],
                                  ssems[b]).wait()

        def transpose(b):
            rows = rows_v.at[b]
            trows = trows_v.at[b]

            @plsc.parallel_loop(0, cb // 16, unroll=2)
            def tbody(j):
                r = lanes + j * 16
                vals = [plsc.load_gather(rows, [r, dcols[d]])
                        for d in range(_D)]
                for d in range(_D):
                    trows[d, pl.ds(j * 16, 16)] = vals[d]

        idx_load(0, 0)
        idx_load(1, 1)
        idx_wait(0, 0)
        gather_start(0)

        def body(j, carry):
            h = 2 * j

            def step(hc, b):
                ob = 1 - b
                @pl.when(hc + 1 < hist)
                def _():
                    idx_wait(hc + 1, ob)
                gather_wait(b)
                @pl.when(hc + 1 < hist)
                def _():
                    gather_start(ob)
                    @pl.when(hc + 2 < hist)
                    def _():
                        idx_load(hc + 2, b)
                @pl.when(hc >= 2)
                def _():
                    store_wait(hc - 2, b)
                transpose(b)
                store_start(hc, b)

            step(h, 0)
            step(h + 1, 1)
            return carry

        lax.fori_loop(0, hist // 2, body, 0)
        store_wait(hist - 2, 0)
        store_wait(hist - 1, 1)

    return k(idx2d, table)


def kernel(x, table):
    b, h = x.shape
    idx2d = jnp.transpose(x.astype(jnp.int32))
    out_t = _sc_gather_t(idx2d, table, h, b)
    return jnp.transpose(out_t, (2, 0, 1))


# speedup vs baseline: 2.1638x; 2.1638x over previous
import functools

import jax
import jax.numpy as jnp
from jax import lax
from jax.experimental import pallas as pl
from jax.experimental.pallas import tpu as pltpu
from jax.experimental.pallas import tpu_sc as plsc

_D = 32
_NC, _NS = 2, 16
_NW = _NC * _NS


@functools.partial(jax.jit, static_argnums=(2, 3))
def _sc_gather_t(idx2d, table, hist, nb):
    cb = nb // _NW
    mesh = plsc.VectorSubcoreMesh(core_axis_name="c", subcore_axis_name="s")

    @functools.partial(
        pl.kernel,
        out_type=jax.ShapeDtypeStruct((hist, _D, nb), jnp.float32),
        mesh=mesh,
        compiler_params=pltpu.CompilerParams(use_tc_tiling_on_sc=False,
                                             needs_layout_passes=False),
        scratch_types=[
            pltpu.VMEM((2, cb), jnp.int32),
            pltpu.VMEM((2, cb, _D), jnp.float32),
            pltpu.VMEM((2, _D, cb), jnp.float32),
            pltpu.SemaphoreType.DMA,
            pltpu.SemaphoreType.DMA,
            pltpu.SemaphoreType.DMA,
            pltpu.SemaphoreType.DMA,
            pltpu.SemaphoreType.DMA,
            pltpu.SemaphoreType.DMA,
        ],
    )
    def k(idx_hbm, table_hbm, out_hbm, idx_v, rows_v, trows_v,
          i0, i1, g0, g1, s0, s1):
        wid = lax.axis_index("s") * _NC + lax.axis_index("c")
        b0 = pl.multiple_of(wid * cb, cb)
        isems = (i0, i1)
        gsems = (g0, g1)
        ssems = (s0, s1)
        lanes = lax.iota(jnp.int32, 16)
        diag = [jnp.bitwise_and(lanes + k, 15) for k in range(16)]

        def idx_load(h, b):
            pltpu.async_copy(idx_hbm.at[h, pl.ds(b0, cb)], idx_v.at[b], isems[b])

        def idx_wait(h, b):
            pltpu.make_async_copy(idx_hbm.at[h, pl.ds(b0, cb)], idx_v.at[b],
                                  isems[b]).wait()

        def gather_start(b):
            pltpu.async_copy(table_hbm.at[idx_v.at[b]], rows_v.at[b], gsems[b])

        def gather_wait(b):
            pltpu.make_async_copy(table_hbm.at[idx_v.at[b]], rows_v.at[b],
                                  gsems[b]).wait()

        def store_start(h, b):
            pltpu.async_copy(trows_v.at[b], out_hbm.at[h, :, pl.ds(b0, cb)],
                             ssems[b])

        def store_wait(h, b):
            pltpu.make_async_copy(trows_v.at[b], out_hbm.at[h, :, pl.ds(b0, cb)],
                                  ssems[b]).wait()

        def transpose(b):
            rows = rows_v.at[b]
            trows = trows_v.at[b]

            @plsc.parallel_loop(0, cb // 16, unroll=2)
            def tbody(j):
                r = lanes + j * 16
                for d0 in range(0, _D, 16):
                    for k in range(16):
                        c = diag[k] + d0
                        v = plsc.load_gather(rows, [r, c])
                        plsc.store_scatter(trows, [c, r], v)

        idx_load(0, 0)
        idx_load(1, 1)
        idx_wait(0, 0)
        gather_start(0)

        def body(j, carry):
            h = 2 * j

            def step(hc, b):
                ob = 1 - b
                @pl.when(hc + 1 < hist)
                def _():
                    idx_wait(hc + 1, ob)
                gather_wait(b)
                @pl.when(hc + 1 < hist)
                def _():
                    gather_start(ob)
                    @pl.when(hc + 2 < hist)
                    def _():
                        idx_load(hc + 2, b)
                @pl.when(hc >= 2)
                def _():
                    store_wait(hc - 2, b)
                transpose(b)
                store_start(hc, b)

            step(h, 0)
            step(h + 1, 1)
            return carry

        lax.fori_loop(0, hist // 2, body, 0)
        store_wait(hist - 2, 0)
        store_wait(hist - 1, 1)

    return k(idx2d, table)


def kernel(x, table):
    b, h = x.shape
    idx2d = jnp.transpose(x.astype(jnp.int32))
    out_t = _sc_gather_t(idx2d, table, h, b)
    return jnp.transpose(out_t, (2, 0, 1))


# speedup vs baseline: 3.1949x; 1.4765x over previous
import functools

import jax
import jax.numpy as jnp
from jax import lax
from jax.experimental import pallas as pl
from jax.experimental.pallas import tpu as pltpu
from jax.experimental.pallas import tpu_sc as plsc

_D = 32
_NC, _NS = 2, 16
_NW = _NC * _NS


@functools.partial(jax.jit, static_argnums=(2, 3))
def _sc_gather_t(idx2d, table, hist, nb):
    cb = nb // _NW
    nkt = _D // 8
    trr = _D * cb // 128
    mesh = plsc.VectorSubcoreMesh(core_axis_name="c", subcore_axis_name="s")

    @functools.partial(
        pl.kernel,
        out_type=jax.ShapeDtypeStruct((hist, nkt, nb // 128 * 8, 128),
                                      jnp.float32),
        mesh=mesh,
        compiler_params=pltpu.CompilerParams(use_tc_tiling_on_sc=False,
                                             needs_layout_passes=False),
        scratch_types=[
            pltpu.VMEM((2, cb), jnp.int32),
            pltpu.VMEM((2, cb, _D), jnp.float32),
            pltpu.VMEM((2, trr, 128), jnp.float32),
            pltpu.SemaphoreType.DMA,
            pltpu.SemaphoreType.DMA,
            pltpu.SemaphoreType.DMA,
            pltpu.SemaphoreType.DMA,
            pltpu.SemaphoreType.DMA,
            pltpu.SemaphoreType.DMA,
        ],
    )
    def k(idx_hbm, table_hbm, out_hbm, idx_v, rows_v, trows_v,
          i0, i1, g0, g1, s0, s1):
        wid = lax.axis_index("s") * _NC + lax.axis_index("c")
        b0 = pl.multiple_of(wid * cb, cb)
        r0row = pl.multiple_of(wid * trr // nkt, trr // nkt)
        isems = (i0, i1)
        gsems = (g0, g1)
        ssems = (s0, s1)
        lanes = lax.iota(jnp.int32, 16)
        diag = [[jnp.bitwise_and(lanes + k, 15) + d0 for k in range(16)]
                for d0 in range(0, _D, 16)]
        rowc = [[(c // 8) * (trr // nkt) + (c % 8) for c in row]
                for row in diag]

        def idx_load(h, b):
            pltpu.async_copy(idx_hbm.at[h, pl.ds(b0, cb)], idx_v.at[b], isems[b])

        def idx_wait(h, b):
            pltpu.make_async_copy(idx_hbm.at[h, pl.ds(b0, cb)], idx_v.at[b],
                                  isems[b]).wait()

        def gather_start(b):
            pltpu.async_copy(table_hbm.at[idx_v.at[b]], rows_v.at[b], gsems[b])

        def gather_wait(b):
            pltpu.make_async_copy(table_hbm.at[idx_v.at[b]], rows_v.at[b],
                                  gsems[b]).wait()

        def store_start(h, b):
            for kt in range(nkt):
                pltpu.async_copy(
                    trows_v.at[b, pl.ds(kt * (trr // nkt), trr // nkt)],
                    out_hbm.at[h, kt, pl.ds(r0row, trr // nkt)], ssems[b])

        def store_wait(h, b):
            for kt in range(nkt):
                pltpu.make_async_copy(
                    trows_v.at[b, pl.ds(kt * (trr // nkt), trr // nkt)],
                    out_hbm.at[h, kt, pl.ds(r0row, trr // nkt)],
                    ssems[b]).wait()

        def transpose(b):
            rows = rows_v.at[b]
            trows = trows_v.at[b]

            @plsc.parallel_loop(0, cb // 16, unroll=2)
            def tbody(j):
                r = lanes + j * 16
                col = lanes + (j % 8) * 16
                m8 = (j // 8) * 8
                for half in range(2):
                    for kd in range(16):
                        v = plsc.load_gather(rows, [r, diag[half][kd]])
                        plsc.store_scatter(trows, [rowc[half][kd] + m8, col], v)

            tbody

        idx_load(0, 0)
        idx_load(1, 1)
        idx_wait(0, 0)
        gather_start(0)

        def body(j, carry):
            h = 2 * j

            def step(hc, b):
                ob = 1 - b
                @pl.when(hc + 1 < hist)
                def _():
                    idx_wait(hc + 1, ob)
                gather_wait(b)
                @pl.when(hc + 1 < hist)
                def _():
                    gather_start(ob)
                    @pl.when(hc + 2 < hist)
                    def _():
                        idx_load(hc + 2, b)
                @pl.when(hc >= 2)
                def _():
                    store_wait(hc - 2, b)
                transpose(b)
                store_start(hc, b)

            step(h, 0)
            step(h + 1, 1)
            return carry

        lax.fori_loop(0, hist // 2, body, 0)
        store_wait(hist - 2, 0)
        store_wait(hist - 1, 1)

    return k(idx2d, table)


def kernel(x, table):
    b, h = x.shape
    idx2d = jnp.transpose(x.astype(jnp.int32))
    out4d = _sc_gather_t(idx2d, table, h, b)
    out5d = out4d.reshape(h, _D // 8, b // 128, 8, 128)
    out = jnp.transpose(out5d, (2, 4, 0, 1, 3)).reshape(b, h, _D)
    return out


# speedup vs baseline: 3.9359x; 1.2319x over previous
import functools

import jax
import jax.numpy as jnp
from jax import lax
from jax.experimental import pallas as pl
from jax.experimental.pallas import tpu as pltpu
from jax.experimental.pallas import tpu_sc as plsc

_D = 32
_NC, _NS = 2, 16
_NW = _NC * _NS


@functools.partial(jax.jit, static_argnums=(2, 3))
def _sc_gather_t(idx2d, table, hist, nb):
    cb = nb // _NW
    nkt = _D // 8
    trr = _D * cb // 128
    mesh = plsc.VectorSubcoreMesh(core_axis_name="c", subcore_axis_name="s")

    @functools.partial(
        pl.kernel,
        out_type=jax.ShapeDtypeStruct((hist, nkt, nb // 128 * 8 * 128),
                                      jnp.float32),
        mesh=mesh,
        compiler_params=pltpu.CompilerParams(use_tc_tiling_on_sc=False,
                                             needs_layout_passes=False),
        scratch_types=[
            pltpu.VMEM((2, cb), jnp.int32),
            pltpu.VMEM((2, cb, _D), jnp.float32),
            pltpu.VMEM((2, nkt, cb * _D // nkt), jnp.float32),
            pltpu.SemaphoreType.DMA,
            pltpu.SemaphoreType.DMA,
            pltpu.SemaphoreType.DMA,
            pltpu.SemaphoreType.DMA,
            pltpu.SemaphoreType.DMA,
            pltpu.SemaphoreType.DMA,
        ],
    )
    def k(idx_hbm, table_hbm, out_hbm, idx_v, rows_v, trows_v,
          i0, i1, g0, g1, s0, s1):
        wid = lax.axis_index("s") * _NC + lax.axis_index("c")
        b0 = pl.multiple_of(wid * cb, cb)
        seg = cb * _D // nkt
        o0 = pl.multiple_of(wid * seg, seg)
        isems = (i0, i1)
        gsems = (g0, g1)
        ssems = (s0, s1)
        lanes = lax.iota(jnp.int32, 16)
        diag = [[jnp.bitwise_and(lanes + k, 15) + d0 for k in range(16)]
                for d0 in range(0, _D, 16)]
        ktv = [[c // 8 for c in row] for row in diag]
        innerb = [[(c % 8) * 128 + lanes for c in row] for row in diag]

        def idx_load(h, b):
            pltpu.async_copy(idx_hbm.at[h, pl.ds(b0, cb)], idx_v.at[b], isems[b])

        def idx_wait(h, b):
            pltpu.make_async_copy(idx_hbm.at[h, pl.ds(b0, cb)], idx_v.at[b],
                                  isems[b]).wait()

        def gather_start(b):
            pltpu.async_copy(table_hbm.at[idx_v.at[b]], rows_v.at[b], gsems[b])

        def gather_wait(b):
            pltpu.make_async_copy(table_hbm.at[idx_v.at[b]], rows_v.at[b],
                                  gsems[b]).wait()

        def store_start(h, b):
            pltpu.async_copy(trows_v.at[b], out_hbm.at[h, :, pl.ds(o0, seg)],
                             ssems[b])

        def store_wait(h, b):
            pltpu.make_async_copy(trows_v.at[b],
                                  out_hbm.at[h, :, pl.ds(o0, seg)],
                                  ssems[b]).wait()

        def transpose(b):
            rows = rows_v.at[b]
            trows = trows_v.at[b]

            @plsc.parallel_loop(0, cb // 16, unroll=2)
            def tbody(j):
                r = lanes + j * 16
                sj = (j // 8) * 1024 + (j % 8) * 16
                for half in range(2):
                    for kd in range(16):
                        v = plsc.load_gather(rows, [r, diag[half][kd]])
                        plsc.store_scatter(
                            trows, [ktv[half][kd], innerb[half][kd] + sj], v)

        idx_load(0, 0)
        idx_load(1, 1)
        idx_wait(0, 0)
        gather_start(0)

        def body(j, carry):
            h = 2 * j

            def step(hc, b):
                ob = 1 - b
                @pl.when(hc + 1 < hist)
                def _():
                    idx_wait(hc + 1, ob)
                gather_wait(b)
                @pl.when(hc + 1 < hist)
                def _():
                    gather_start(ob)
                    @pl.when(hc + 2 < hist)
                    def _():
                        idx_load(hc + 2, b)
                @pl.when(hc >= 2)
                def _():
                    store_wait(hc - 2, b)
                transpose(b)
                store_start(hc, b)

            step(h, 0)
            step(h + 1, 1)
            return carry

        lax.fori_loop(0, hist // 2, body, 0)
        store_wait(hist - 2, 0)
        store_wait(hist - 1, 1)

    return k(idx2d, table)


def kernel(x, table):
    b, h = x.shape
    idx2d = jnp.transpose(x.astype(jnp.int32))
    out3d = _sc_gather_t(idx2d, table, h, b)
    out5d = out3d.reshape(h, _D // 8, b // 128, 8, 128)
    out = jnp.transpose(out5d, (2, 4, 0, 1, 3)).reshape(b, h, _D)
    return out


# speedup vs baseline: 4.2024x; 1.0677x over previous
import functools

import jax
import jax.numpy as jnp
from jax import lax
from jax.experimental import pallas as pl
from jax.experimental.pallas import tpu as pltpu
from jax.experimental.pallas import tpu_sc as plsc

_D = 32
_NC, _NS = 2, 16
_NW = _NC * _NS


@functools.partial(jax.jit, static_argnums=(2, 3))
def _sc_gather_t(idx2d, table, hist, nb):
    cb = nb // _NW
    nkt = _D // 8
    trr = _D * cb // 128
    mesh = plsc.VectorSubcoreMesh(core_axis_name="c", subcore_axis_name="s")

    @functools.partial(
        pl.kernel,
        out_type=jax.ShapeDtypeStruct((hist, nkt, nb // 128 * 8 * 128),
                                      jnp.float32),
        mesh=mesh,
        compiler_params=pltpu.CompilerParams(use_tc_tiling_on_sc=False,
                                             needs_layout_passes=False),
        scratch_types=[
            pltpu.VMEM((2, cb), jnp.int32),
            pltpu.VMEM((2, cb, _D), jnp.float32),
            pltpu.VMEM((2, nkt, cb * _D // nkt), jnp.float32),
            pltpu.SemaphoreType.DMA,
            pltpu.SemaphoreType.DMA,
            pltpu.SemaphoreType.DMA,
            pltpu.SemaphoreType.DMA,
            pltpu.SemaphoreType.DMA,
            pltpu.SemaphoreType.DMA,
        ],
    )
    def k(idx_hbm, table_hbm, out_hbm, idx_v, rows_v, trows_v,
          i0, i1, g0, g1, s0, s1):
        wid = lax.axis_index("s") * _NC + lax.axis_index("c")
        b0 = pl.multiple_of(wid * cb, cb)
        seg = cb * _D // nkt
        o0 = pl.multiple_of(wid * seg, seg)
        isems = (i0, i1)
        gsems = (g0, g1)
        ssems = (s0, s1)
        lanes = lax.iota(jnp.int32, 16)
        diag = [[jnp.bitwise_and(lanes + k, 15) + d0 for k in range(16)]
                for d0 in range(0, _D, 16)]
        ktv = [[c // 8 for c in row] for row in diag]
        innerb = [[(c % 8) * 128 + lanes for c in row] for row in diag]

        def idx_load(h, b):
            pltpu.async_copy(idx_hbm.at[h, pl.ds(b0, cb)], idx_v.at[b], isems[b])

        def idx_wait(h, b):
            pltpu.make_async_copy(idx_hbm.at[h, pl.ds(b0, cb)], idx_v.at[b],
                                  isems[b]).wait()

        def gather_start(b):
            pltpu.async_copy(table_hbm.at[idx_v.at[b]], rows_v.at[b], gsems[b])

        def gather_wait(b):
            pltpu.make_async_copy(table_hbm.at[idx_v.at[b]], rows_v.at[b],
                                  gsems[b]).wait()

        def store_start(h, b):
            pltpu.async_copy(trows_v.at[b], out_hbm.at[h, :, pl.ds(o0, seg)],
                             ssems[b])

        def store_wait(h, b):
            pltpu.make_async_copy(trows_v.at[b],
                                  out_hbm.at[h, :, pl.ds(o0, seg)],
                                  ssems[b]).wait()

        def transpose(b):
            rows = rows_v.at[b]
            trows = trows_v.at[b]

            @plsc.parallel_loop(0, cb // 16, unroll=4)
            def tbody(j):
                r = lanes + j * 16
                sj = (j // 8) * 1024 + (j % 8) * 16
                for half in range(2):
                    for kd in range(16):
                        v = plsc.load_gather(rows, [r, diag[half][kd]])
                        plsc.store_scatter(
                            trows, [ktv[half][kd], innerb[half][kd] + sj], v)

        idx_load(0, 0)
        idx_load(1, 1)
        idx_wait(0, 0)
        gather_start(0)

        def body(j, carry):
            h = 2 * j

            def step(hc, b):
                ob = 1 - b
                @pl.when(hc + 1 < hist)
                def _():
                    idx_wait(hc + 1, ob)
                gather_wait(b)
                @pl.when(hc + 1 < hist)
                def _():
                    gather_start(ob)
                    @pl.when(hc + 2 < hist)
                    def _():
                        idx_load(hc + 2, b)
                @pl.when(hc >= 2)
                def _():
                    store_wait(hc - 2, b)
                transpose(b)
                store_start(hc, b)

            step(h, 0)
            step(h + 1, 1)
            return carry

        lax.fori_loop(0, hist // 2, body, 0)
        store_wait(hist - 2, 0)
        store_wait(hist - 1, 1)

    return k(idx2d, table)


def kernel(x, table):
    b, h = x.shape
    idx2d = jnp.transpose(x.astype(jnp.int32))
    out3d = _sc_gather_t(idx2d, table, h, b)
    out5d = out3d.reshape(h, _D // 8, b // 128, 8, 128)
    out = jnp.transpose(out5d, (2, 4, 0, 1, 3)).reshape(b, h, _D)
    return out
